# baseline (speedup 1.0000x reference)
import functools

import jax
import jax.numpy as jnp
from jax import lax
from jax.experimental import pallas as pl
from jax.experimental.pallas import tpu as pltpu
from jax.experimental.pallas import tpu_sc as plsc

B = 16
S = 512
KK = 64
N = 64
T = 64
D = N * T
M = B * N * T
NW = 32
CM = 64
NCH = (D // 2) // CM
A_ROWS_PER_W = (B * KK) // NW
A_WORDS_PER_W = A_ROWS_PER_W * S


@functools.partial(
    pl.kernel,
    out_type=jax.ShapeDtypeStruct((M // 2, 128), jnp.float32),
    mesh=plsc.VectorSubcoreMesh(core_axis_name="c", subcore_axis_name="s"),
    compiler_params=pltpu.CompilerParams(needs_layout_passes=False),
    scratch_types=[
        pltpu.VMEM((B * KK,), jnp.int32),
        pltpu.VMEM((CM, S), jnp.float32),
        pltpu.VMEM((CM, S), jnp.float32),
        pltpu.VMEM((CM, S), jnp.float32),
        pltpu.VMEM((KK, 128), jnp.float32),
        pltpu.VMEM((KK, 128), jnp.float32),
        pltpu.VMEM((KK,), jnp.int32),
        pltpu.VMEM((KK,), jnp.int32),
        pltpu.SemaphoreType.DMA,
        pltpu.SemaphoreType.DMA,
        pltpu.SemaphoreType.DMA,
        pltpu.SemaphoreType.DMA,
        pltpu.SemaphoreType.DMA,
    ],
)
def _sc_filter(y_hbm, idx_hbm, ysel_hbm,
               idx_v, in0, in1, in2, out0, out1, rid0, rid1,
               gs0, gs1, gs2, os0, os1):
    w = lax.axis_index("s") * 2 + lax.axis_index("c")
    b = w // 2
    half = w % 2
    lane = lax.broadcasted_iota(jnp.int32, (16,), 0)
    zero16 = jnp.zeros((16,), jnp.int32)

    pltpu.sync_copy(idx_hbm, idx_v)

    mbase = b * D + half * (D // 2)
    inb = (in0, in1, in2)
    outb = (out0, out1)
    ridb = (rid0, rid1)
    gsem = (gs0, gs1, gs2)
    osem = (os0, os1)

    def copy_in(ch, p):
        return pltpu.async_copy(
            y_hbm.at[pl.ds(mbase + ch * CM, CM)], inb[p], gsem[p])

    gin = [copy_in(0, 0), copy_in(1, 1), copy_in(2, 2)]

    kidx = []
    kvec = []
    for j in range(4):
        kidx.append(plsc.load_gather(idx_v, [b * KK + j * 16 + lane]))
        kvec.append(j * 16 + lane)

    rbase = b * 32 + half * (NCH // 2)

    gout = [None, None]
    for ch in range(NCH):
        p = ch % 3
        q = ch // 2
        qp = q % 2
        if ch % 2 == 0 and gout[qp] is not None:
            gout[qp].wait()
        gin[p].wait()

        src = inb[p]
        dst = outb[qp]
        coff = (ch % 2) * CM

        @plsc.parallel_loop(0, KK, 1, unroll=8)
        def _extract(k, src=src, dst=dst, coff=coff):
            sk = plsc.load_gather(idx_v, [zero16 + (b * KK + k)])
            for mg in range(4):
                v = plsc.load_gather(src, [lane + mg * 16, sk])
                dst[k, pl.ds(coff + mg * 16, 16)] = v

        if ch % 2 == 1:
            rv = q + rbase
            for j in range(4):
                ridb[qp][pl.ds(j * 16, 16)] = kvec[j] * 512 + rv
            gout[qp] = pltpu.async_copy(
                outb[qp], ysel_hbm.at[ridb[qp]], osem[qp])
        if ch + 3 < NCH:
            gin[p] = copy_in(ch + 3, p)

    gout[0].wait()
    gout[1].wait()


def _a_onehot_body(idx_ref, a0_ref, a1_ref):
    iota_s = lax.broadcasted_iota(jnp.int32, (B, KK, S), 2)
    hit = idx_ref[...][:, :, None] == iota_s
    oh = jnp.where(hit, 1.0, 0.0).astype(jnp.float32)
    a0_ref[...] = oh
    a1_ref[...] = oh


_a_onehot = pl.pallas_call(
    _a_onehot_body,
    out_shape=(
        jax.ShapeDtypeStruct((B, KK, S), jnp.float32),
        jax.ShapeDtypeStruct((B, KK, S), jnp.float32),
    ),
)


def kernel(Y_full, idx_all):
    y_t = jnp.transpose(Y_full, (0, 2, 3, 1)).reshape(M, S)
    idx_flat = idx_all.reshape(-1)
    ysel_fine = _sc_filter(y_t, idx_flat)
    A, A2 = _a_onehot(idx_all)
    Y_sel = ysel_fine.reshape(KK, B, N, T)
    return (Y_sel, A, A2)


# speedup vs baseline: 1.0490x; 1.0490x over previous
import functools

import jax
import jax.numpy as jnp
from jax import lax
from jax.experimental import pallas as pl
from jax.experimental.pallas import tpu as pltpu
from jax.experimental.pallas import tpu_sc as plsc

B = 16
S = 512
KK = 64
N = 64
T = 64
D = N * T
M = B * N * T
NW = 32
CM = 64
NCH = (D // 2) // CM
A_ROWS_PER_W = (B * KK) // NW
A_WORDS_PER_W = A_ROWS_PER_W * S


@functools.partial(
    pl.kernel,
    out_type=jax.ShapeDtypeStruct((M // 2, 128), jnp.float32),
    mesh=plsc.VectorSubcoreMesh(core_axis_name="c", subcore_axis_name="s"),
    compiler_params=pltpu.CompilerParams(needs_layout_passes=False),
    scratch_types=[
        pltpu.VMEM((B * KK,), jnp.int32),
        pltpu.VMEM((CM, S), jnp.float32),
        pltpu.VMEM((CM, S), jnp.float32),
        pltpu.VMEM((CM, S), jnp.float32),
        pltpu.VMEM((KK, 128), jnp.float32),
        pltpu.VMEM((KK, 128), jnp.float32),
        pltpu.VMEM((KK,), jnp.int32),
        pltpu.VMEM((KK,), jnp.int32),
        pltpu.VMEM((KK, 16), jnp.int32),
        pltpu.SemaphoreType.DMA,
        pltpu.SemaphoreType.DMA,
        pltpu.SemaphoreType.DMA,
        pltpu.SemaphoreType.DMA,
        pltpu.SemaphoreType.DMA,
    ],
)
def _sc_filter(y_hbm, idx_hbm, ysel_hbm,
               idx_v, in0, in1, in2, out0, out1, rid0, rid1, sidx,
               gs0, gs1, gs2, os0, os1):
    w = lax.axis_index("s") * 2 + lax.axis_index("c")
    b = w // 2
    half = w % 2
    lane = lax.broadcasted_iota(jnp.int32, (16,), 0)
    zero16 = jnp.zeros((16,), jnp.int32)

    pltpu.sync_copy(idx_hbm, idx_v)

    mbase = b * D + half * (D // 2)
    inb = (in0, in1, in2)
    outb = (out0, out1)
    ridb = (rid0, rid1)
    gsem = (gs0, gs1, gs2)
    osem = (os0, os1)

    def copy_in(ch, p):
        return pltpu.async_copy(
            y_hbm.at[pl.ds(mbase + ch * CM, CM)], inb[p], gsem[p])

    gin = [copy_in(0, 0), copy_in(1, 1), copy_in(2, 2)]

    kvec = [j * 16 + lane for j in range(4)]

    @plsc.parallel_loop(0, KK, 1, unroll=4)
    def _mksidx(k):
        sidx[k] = plsc.load_gather(idx_v, [zero16 + (b * KK + k)])

    rbase = b * 32 + half * (NCH // 2)

    gout = [None, None]
    for ch in range(NCH):
        p = ch % 3
        q = ch // 2
        qp = q % 2
        if ch % 2 == 0 and gout[qp] is not None:
            gout[qp].wait()
        gin[p].wait()

        src = inb[p]
        dst = outb[qp]
        coff = (ch % 2) * CM

        @plsc.parallel_loop(0, KK, 1, unroll=4)
        def _extract(k, src=src, dst=dst, coff=coff):
            sk = sidx[k]
            for mg in range(4):
                v = plsc.load_gather(src, [lane + mg * 16, sk])
                dst[k, pl.ds(coff + mg * 16, 16)] = v

        if ch % 2 == 1:
            rv = q + rbase
            for j in range(4):
                ridb[qp][pl.ds(j * 16, 16)] = kvec[j] * 512 + rv
            gout[qp] = pltpu.async_copy(
                outb[qp], ysel_hbm.at[ridb[qp]], osem[qp])
        if ch + 3 < NCH:
            gin[p] = copy_in(ch + 3, p)

    gout[0].wait()
    gout[1].wait()


def _a_onehot_body(idx_ref, a0_ref, a1_ref):
    iota_s = lax.broadcasted_iota(jnp.int32, (B, KK, S), 2)
    hit = idx_ref[...][:, :, None] == iota_s
    oh = jnp.where(hit, 1.0, 0.0).astype(jnp.float32)
    a0_ref[...] = oh
    a1_ref[...] = oh


_a_onehot = pl.pallas_call(
    _a_onehot_body,
    out_shape=(
        jax.ShapeDtypeStruct((B, KK, S), jnp.float32),
        jax.ShapeDtypeStruct((B, KK, S), jnp.float32),
    ),
)


def kernel(Y_full, idx_all):
    y_t = jnp.transpose(Y_full, (0, 2, 3, 1)).reshape(M, S)
    idx_flat = idx_all.reshape(-1)
    ysel_fine = _sc_filter(y_t, idx_flat)
    A, A2 = _a_onehot(idx_all)
    Y_sel = ysel_fine.reshape(KK, B, N, T)
    return (Y_sel, A, A2)


# speedup vs baseline: 1.0501x; 1.0011x over previous
import functools

import jax
import jax.numpy as jnp
from jax import lax
from jax.experimental import pallas as pl
from jax.experimental.pallas import tpu as pltpu
from jax.experimental.pallas import tpu_sc as plsc

B = 16
S = 512
KK = 64
N = 64
T = 64
D = N * T
M = B * N * T
NW = 32
CM = 64
NCH = (D // 2) // CM
A_ROWS_PER_W = (B * KK) // NW
A_WORDS_PER_W = A_ROWS_PER_W * S


@functools.partial(
    pl.kernel,
    out_type=jax.ShapeDtypeStruct((M // 2, 128), jnp.float32),
    mesh=plsc.VectorSubcoreMesh(core_axis_name="c", subcore_axis_name="s"),
    compiler_params=pltpu.CompilerParams(needs_layout_passes=False, skip_device_barrier=True),
    scratch_types=[
        pltpu.VMEM((B * KK,), jnp.int32),
        pltpu.VMEM((CM, S), jnp.float32),
        pltpu.VMEM((CM, S), jnp.float32),
        pltpu.VMEM((CM, S), jnp.float32),
        pltpu.VMEM((KK, 128), jnp.float32),
        pltpu.VMEM((KK, 128), jnp.float32),
        pltpu.VMEM((KK,), jnp.int32),
        pltpu.VMEM((KK,), jnp.int32),
        pltpu.VMEM((KK, 16), jnp.int32),
        pltpu.SemaphoreType.DMA,
        pltpu.SemaphoreType.DMA,
        pltpu.SemaphoreType.DMA,
        pltpu.SemaphoreType.DMA,
        pltpu.SemaphoreType.DMA,
    ],
)
def _sc_filter(y_hbm, idx_hbm, ysel_hbm,
               idx_v, in0, in1, in2, out0, out1, rid0, rid1, sidx,
               gs0, gs1, gs2, os0, os1):
    w = lax.axis_index("s") * 2 + lax.axis_index("c")
    b = w // 2
    half = w % 2
    lane = lax.broadcasted_iota(jnp.int32, (16,), 0)
    zero16 = jnp.zeros((16,), jnp.int32)

    pltpu.sync_copy(idx_hbm, idx_v)

    mbase = b * D + half * (D // 2)
    inb = (in0, in1, in2)
    outb = (out0, out1)
    ridb = (rid0, rid1)
    gsem = (gs0, gs1, gs2)
    osem = (os0, os1)

    def copy_in(ch, p):
        return pltpu.async_copy(
            y_hbm.at[pl.ds(mbase + ch * CM, CM)], inb[p], gsem[p])

    gin = [copy_in(0, 0), copy_in(1, 1), copy_in(2, 2)]

    kvec = [j * 16 + lane for j in range(4)]

    @plsc.parallel_loop(0, KK, 1, unroll=4)
    def _mksidx(k):
        sidx[k] = plsc.load_gather(idx_v, [zero16 + (b * KK + k)])

    rbase = b * 32 + half * (NCH // 2)

    gout = [None, None]
    for ch in range(NCH):
        p = ch % 3
        q = ch // 2
        qp = q % 2
        if ch % 2 == 0 and gout[qp] is not None:
            gout[qp].wait()
        gin[p].wait()

        src = inb[p]
        dst = outb[qp]
        coff = (ch % 2) * CM

        @plsc.parallel_loop(0, KK, 1, unroll=4)
        def _extract(k, src=src, dst=dst, coff=coff):
            sk = sidx[k]
            for mg in range(4):
                v = plsc.load_gather(src, [lane + mg * 16, sk])
                dst[k, pl.ds(coff + mg * 16, 16)] = v

        if ch % 2 == 1:
            rv = q + rbase
            for j in range(4):
                ridb[qp][pl.ds(j * 16, 16)] = kvec[j] * 512 + rv
            gout[qp] = pltpu.async_copy(
                outb[qp], ysel_hbm.at[ridb[qp]], osem[qp])
        if ch + 3 < NCH:
            gin[p] = copy_in(ch + 3, p)

    gout[0].wait()
    gout[1].wait()


def _a_onehot_body(idx_ref, a0_ref, a1_ref):
    iota_s = lax.broadcasted_iota(jnp.int32, (B, KK, S), 2)
    hit = idx_ref[...][:, :, None] == iota_s
    oh = jnp.where(hit, 1.0, 0.0).astype(jnp.float32)
    a0_ref[...] = oh
    a1_ref[...] = oh


_a_onehot = pl.pallas_call(
    _a_onehot_body,
    out_shape=(
        jax.ShapeDtypeStruct((B, KK, S), jnp.float32),
        jax.ShapeDtypeStruct((B, KK, S), jnp.float32),
    ),
)


def kernel(Y_full, idx_all):
    y_t = jnp.transpose(Y_full, (0, 2, 3, 1)).reshape(M, S)
    idx_flat = idx_all.reshape(-1)
    ysel_fine = _sc_filter(y_t, idx_flat)
    A, A2 = _a_onehot(idx_all)
    Y_sel = ysel_fine.reshape(KK, B, N, T)
    return (Y_sel, A, A2)


# speedup vs baseline: 1.0584x; 1.0079x over previous
import functools

import jax
import jax.numpy as jnp
from jax import lax
from jax.experimental import pallas as pl
from jax.experimental.pallas import tpu as pltpu
from jax.experimental.pallas import tpu_sc as plsc

B = 16
S = 512
KK = 64
N = 64
T = 64
D = N * T
M = B * N * T
NW = 32
CM = 64
NCH = (D // 2) // CM
A_ROWS_PER_W = (B * KK) // NW
A_WORDS_PER_W = A_ROWS_PER_W * S


@functools.partial(
    pl.kernel,
    out_type=jax.ShapeDtypeStruct((M // 2, 128), jnp.float32),
    mesh=plsc.VectorSubcoreMesh(core_axis_name="c", subcore_axis_name="s"),
    compiler_params=pltpu.CompilerParams(needs_layout_passes=False),
    scratch_types=[
        pltpu.VMEM((B * KK,), jnp.int32),
        pltpu.VMEM((CM, S), jnp.float32),
        pltpu.VMEM((CM, S), jnp.float32),
        pltpu.VMEM((CM, S), jnp.float32),
        pltpu.VMEM((KK, 128), jnp.float32),
        pltpu.VMEM((KK, 128), jnp.float32),
        pltpu.VMEM((KK,), jnp.int32),
        pltpu.VMEM((KK,), jnp.int32),
        pltpu.VMEM((KK, 16), jnp.int32),
        pltpu.SemaphoreType.DMA,
        pltpu.SemaphoreType.DMA,
        pltpu.SemaphoreType.DMA,
        pltpu.SemaphoreType.DMA,
        pltpu.SemaphoreType.DMA,
    ],
)
def _sc_filter(y_hbm, idx_hbm, ysel_hbm,
               idx_v, in0, in1, in2, out0, out1, rid0, rid1, sidx,
               gs0, gs1, gs2, os0, os1):
    w = lax.axis_index("s") * 2 + lax.axis_index("c")
    b = w // 2
    half = w % 2
    lane = lax.broadcasted_iota(jnp.int32, (16,), 0)
    zero16 = jnp.zeros((16,), jnp.int32)

    pltpu.sync_copy(idx_hbm, idx_v)

    mbase = b * D + half * (D // 2)
    inb = (in0, in1, in2)
    outb = (out0, out1)
    ridb = (rid0, rid1)
    gsem = (gs0, gs1, gs2)
    osem = (os0, os1)

    def copy_in(ch, p):
        return pltpu.async_copy(
            y_hbm.at[pl.ds(mbase + ch * CM, CM)], inb[p], gsem[p])

    gin = [copy_in(0, 0), copy_in(1, 1), copy_in(2, 2)]

    kvec = [j * 16 + lane for j in range(4)]

    @plsc.parallel_loop(0, KK, 1, unroll=4)
    def _mksidx(k):
        sidx[k] = plsc.load_gather(idx_v, [zero16 + (b * KK + k)])

    rbase = b * 32 + half * (NCH // 2)

    gout = [None, None]
    for ch in range(NCH):
        p = ch % 3
        q = ch // 2
        qp = q % 2
        if ch % 2 == 0 and gout[qp] is not None:
            gout[qp].wait()
        gin[p].wait()

        src = inb[p]
        dst = outb[qp]
        coff = (ch % 2) * CM

        @plsc.parallel_loop(0, KK, 1, unroll=2)
        def _extract(k, src=src, dst=dst, coff=coff):
            sk = sidx[k]
            for mg in range(4):
                v = plsc.load_gather(src, [lane + mg * 16, sk])
                dst[k, pl.ds(coff + mg * 16, 16)] = v

        if ch % 2 == 1:
            rv = q + rbase
            for j in range(4):
                ridb[qp][pl.ds(j * 16, 16)] = kvec[j] * 512 + rv
            gout[qp] = pltpu.async_copy(
                outb[qp], ysel_hbm.at[ridb[qp]], osem[qp])
        if ch + 3 < NCH:
            gin[p] = copy_in(ch + 3, p)

    gout[0].wait()
    gout[1].wait()


def _a_onehot_body(idx_ref, a0_ref, a1_ref):
    iota_s = lax.broadcasted_iota(jnp.int32, (B, KK, S), 2)
    hit = idx_ref[...][:, :, None] == iota_s
    oh = jnp.where(hit, 1.0, 0.0).astype(jnp.float32)
    a0_ref[...] = oh
    a1_ref[...] = oh


_a_onehot = pl.pallas_call(
    _a_onehot_body,
    out_shape=(
        jax.ShapeDtypeStruct((B, KK, S), jnp.float32),
        jax.ShapeDtypeStruct((B, KK, S), jnp.float32),
    ),
)


def kernel(Y_full, idx_all):
    y_t = jnp.transpose(Y_full, (0, 2, 3, 1)).reshape(M, S)
    idx_flat = idx_all.reshape(-1)
    ysel_fine = _sc_filter(y_t, idx_flat)
    A, A2 = _a_onehot(idx_all)
    Y_sel = ysel_fine.reshape(KK, B, N, T)
    return (Y_sel, A, A2)


# speedup vs baseline: 1.0652x; 1.0064x over previous
import functools

import jax
import jax.numpy as jnp
from jax import lax
from jax.experimental import pallas as pl
from jax.experimental.pallas import tpu as pltpu
from jax.experimental.pallas import tpu_sc as plsc

B = 16
S = 512
KK = 64
N = 64
T = 64
D = N * T
M = B * N * T
NW = 32
CM = 64
NCH = (D // 2) // CM
A_ROWS_PER_W = (B * KK) // NW
A_WORDS_PER_W = A_ROWS_PER_W * S


@functools.partial(
    pl.kernel,
    out_type=jax.ShapeDtypeStruct((M // 2, 128), jnp.float32),
    mesh=plsc.VectorSubcoreMesh(core_axis_name="c", subcore_axis_name="s"),
    compiler_params=pltpu.CompilerParams(needs_layout_passes=False),
    scratch_types=[
        pltpu.VMEM((B * KK,), jnp.int32),
        pltpu.VMEM((CM, S), jnp.float32),
        pltpu.VMEM((CM, S), jnp.float32),
        pltpu.VMEM((CM, S), jnp.float32),
        pltpu.VMEM((KK, 128), jnp.float32),
        pltpu.VMEM((KK, 128), jnp.float32),
        pltpu.VMEM((KK,), jnp.int32),
        pltpu.VMEM((KK,), jnp.int32),
        pltpu.VMEM((KK, 16), jnp.int32),
        pltpu.SemaphoreType.DMA,
        pltpu.SemaphoreType.DMA,
        pltpu.SemaphoreType.DMA,
        pltpu.SemaphoreType.DMA,
        pltpu.SemaphoreType.DMA,
    ],
)
def _sc_filter(y_hbm, idx_hbm, ysel_hbm,
               idx_v, in0, in1, in2, out0, out1, rid0, rid1, sidx,
               gs0, gs1, gs2, os0, os1):
    w = lax.axis_index("s") * 2 + lax.axis_index("c")
    b = w // 2
    half = w % 2
    lane = lax.broadcasted_iota(jnp.int32, (16,), 0)
    zero16 = jnp.zeros((16,), jnp.int32)

    mbase = b * D + half * (D // 2)
    inb = (in0, in1, in2)
    outb = (out0, out1)
    ridb = (rid0, rid1)
    gsem = (gs0, gs1, gs2)
    osem = (os0, os1)

    def copy_in(ch, p):
        return pltpu.async_copy(
            y_hbm.at[pl.ds(mbase + ch * CM, CM)], inb[p], gsem[p])

    gin = [copy_in(0, 0), copy_in(1, 1), copy_in(2, 2)]

    pltpu.sync_copy(idx_hbm, idx_v)

    kvec = [j * 16 + lane for j in range(4)]

    @plsc.parallel_loop(0, KK, 1, unroll=4)
    def _mksidx(k):
        sidx[k] = plsc.load_gather(idx_v, [zero16 + (b * KK + k)])

    rbase = b * 32 + half * (NCH // 2)

    gout = [None, None]
    for ch in range(NCH):
        p = ch % 3
        q = ch // 2
        qp = q % 2
        if ch % 2 == 0 and gout[qp] is not None:
            gout[qp].wait()
        gin[p].wait()

        src = inb[p]
        dst = outb[qp]
        coff = (ch % 2) * CM

        @plsc.parallel_loop(0, KK, 1, unroll=2)
        def _extract(k, src=src, dst=dst, coff=coff):
            sk = sidx[k]
            for mg in range(4):
                v = plsc.load_gather(src, [lane + mg * 16, sk])
                dst[k, pl.ds(coff + mg * 16, 16)] = v

        if ch % 2 == 1:
            rv = q + rbase
            for j in range(4):
                ridb[qp][pl.ds(j * 16, 16)] = kvec[j] * 512 + rv
            gout[qp] = pltpu.async_copy(
                outb[qp], ysel_hbm.at[ridb[qp]], osem[qp])
        if ch + 3 < NCH:
            gin[p] = copy_in(ch + 3, p)

    gout[0].wait()
    gout[1].wait()


def _a_onehot_body(idx_ref, a0_ref, a1_ref):
    iota_s = lax.broadcasted_iota(jnp.int32, (B, KK, S), 2)
    hit = idx_ref[...][:, :, None] == iota_s
    oh = jnp.where(hit, 1.0, 0.0).astype(jnp.float32)
    a0_ref[...] = oh
    a1_ref[...] = oh


_a_onehot = pl.pallas_call(
    _a_onehot_body,
    out_shape=(
        jax.ShapeDtypeStruct((B, KK, S), jnp.float32),
        jax.ShapeDtypeStruct((B, KK, S), jnp.float32),
    ),
)


def kernel(Y_full, idx_all):
    y_t = jnp.transpose(Y_full, (0, 2, 3, 1)).reshape(M, S)
    idx_flat = idx_all.reshape(-1)
    ysel_fine = _sc_filter(y_t, idx_flat)
    A, A2 = _a_onehot(idx_all)
    Y_sel = ysel_fine.reshape(KK, B, N, T)
    return (Y_sel, A, A2)


# speedup vs baseline: 1.0759x; 1.0101x over previous
import functools

import jax
import jax.numpy as jnp
from jax import lax
from jax.experimental import pallas as pl
from jax.experimental.pallas import tpu as pltpu
from jax.experimental.pallas import tpu_sc as plsc

B = 16
S = 512
KK = 64
N = 64
T = 64
D = N * T
M = B * N * T
NW = 32
CM = 64
NCH = (D // 2) // CM
A_ROWS_PER_W = (B * KK) // NW
A_WORDS_PER_W = A_ROWS_PER_W * S


@functools.partial(
    pl.kernel,
    out_type=jax.ShapeDtypeStruct((M // 2, 128), jnp.float32),
    mesh=plsc.VectorSubcoreMesh(core_axis_name="c", subcore_axis_name="s"),
    compiler_params=pltpu.CompilerParams(needs_layout_passes=False),
    scratch_types=[
        pltpu.VMEM((B * KK,), jnp.int32),
        pltpu.VMEM((CM, S), jnp.float32),
        pltpu.VMEM((CM, S), jnp.float32),
        pltpu.VMEM((CM, S), jnp.float32),
        pltpu.VMEM((KK, 128), jnp.float32),
        pltpu.VMEM((KK, 128), jnp.float32),
        pltpu.VMEM((KK,), jnp.int32),
        pltpu.VMEM((KK,), jnp.int32),
        pltpu.VMEM((KK, 16), jnp.int32),
        pltpu.SemaphoreType.DMA,
        pltpu.SemaphoreType.DMA,
        pltpu.SemaphoreType.DMA,
        pltpu.SemaphoreType.DMA,
        pltpu.SemaphoreType.DMA,
    ],
)
def _sc_filter(y_hbm, idx_hbm, ysel_hbm,
               idx_v, in0, in1, in2, out0, out1, rid0, rid1, sidx,
               gs0, gs1, gs2, os0, os1):
    w = lax.axis_index("s") * 2 + lax.axis_index("c")
    b = w // 2
    half = w % 2
    lane = lax.broadcasted_iota(jnp.int32, (16,), 0)
    zero16 = jnp.zeros((16,), jnp.int32)

    mbase = b * D + half * (D // 2)
    inb = (in0, in1, in2)
    outb = (out0, out1)
    ridb = (rid0, rid1)
    gsem = (gs0, gs1, gs2)
    osem = (os0, os1)

    def copy_in(ch, p):
        return pltpu.async_copy(
            y_hbm.at[pl.ds(mbase + ch * CM, CM)], inb[p], gsem[p])

    gin = [copy_in(0, 0), copy_in(1, 1), copy_in(2, 2)]

    pltpu.sync_copy(idx_hbm, idx_v)

    kvec = [j * 16 + lane for j in range(4)]

    @plsc.parallel_loop(0, KK, 1, unroll=4)
    def _mksidx(k):
        sidx[k] = plsc.load_gather(idx_v, [zero16 + (b * KK + k)])

    rbase = b * 32 + half * (NCH // 2)

    gout = [None, None]
    for ch in range(NCH):
        p = ch % 3
        q = ch // 2
        qp = q % 2
        if ch % 2 == 0 and gout[qp] is not None:
            gout[qp].wait()
        gin[p].wait()

        src = inb[p]
        dst = outb[qp]
        coff = (ch % 2) * CM

        @plsc.parallel_loop(0, KK, 1, unroll=1)
        def _extract(k, src=src, dst=dst, coff=coff):
            sk = sidx[k]
            for mg in range(4):
                v = plsc.load_gather(src, [lane + mg * 16, sk])
                dst[k, pl.ds(coff + mg * 16, 16)] = v

        if ch % 2 == 1:
            rv = q + rbase
            for j in range(4):
                ridb[qp][pl.ds(j * 16, 16)] = kvec[j] * 512 + rv
            gout[qp] = pltpu.async_copy(
                outb[qp], ysel_hbm.at[ridb[qp]], osem[qp])
        if ch + 3 < NCH:
            gin[p] = copy_in(ch + 3, p)

    gout[0].wait()
    gout[1].wait()


def _a_onehot_body(idx_ref, a0_ref, a1_ref):
    iota_s = lax.broadcasted_iota(jnp.int32, (B, KK, S), 2)
    hit = idx_ref[...][:, :, None] == iota_s
    oh = jnp.where(hit, 1.0, 0.0).astype(jnp.float32)
    a0_ref[...] = oh
    a1_ref[...] = oh


_a_onehot = pl.pallas_call(
    _a_onehot_body,
    out_shape=(
        jax.ShapeDtypeStruct((B, KK, S), jnp.float32),
        jax.ShapeDtypeStruct((B, KK, S), jnp.float32),
    ),
)


def kernel(Y_full, idx_all):
    y_t = jnp.transpose(Y_full, (0, 2, 3, 1)).reshape(M, S)
    idx_flat = idx_all.reshape(-1)
    ysel_fine = _sc_filter(y_t, idx_flat)
    A, A2 = _a_onehot(idx_all)
    Y_sel = ysel_fine.reshape(KK, B, N, T)
    return (Y_sel, A, A2)
